# baseline (speedup 1.0000x reference)
import jax
import jax.numpy as jnp
from jax.experimental import pallas as pl
from jax.experimental.pallas import tpu as pltpu

D = 2048
N_EXP = 64
TOP_K = 8
N_TOK = 16384

TB = 2048
DC = 1024
ND = D // DC


def _router_block(x_ref, w_ref, vals_ref, inds_ref, gates_ref, acc_ref):
    j = pl.program_id(1)
    partial = jax.lax.dot_general(
        x_ref[...], w_ref[...], (((1,), (1,)), ((), ())),
        preferred_element_type=jnp.float32,
    )

    @pl.when(j == 0)
    def _init():
        acc_ref[...] = partial

    @pl.when(j > 0)
    def _accum():
        acc_ref[...] += partial

    @pl.when(j == ND - 1)
    def _finish():
        logits = acc_ref[...]
        lt = logits.T
        m = jnp.max(lt, axis=0, keepdims=True)
        e = jnp.exp(lt - m)
        s = jnp.sum(e, axis=0, keepdims=True)
        gt = e / s
        gates_ref[...] = gt.T

        iota = jax.lax.broadcasted_iota(jnp.int32, (N_EXP, TB), 0)
        work = gt
        vals_rows = []
        inds_rows = []
        for _ in range(TOP_K):
            mx = jnp.max(work, axis=0, keepdims=True)
            idx = jnp.min(
                jnp.where(work == mx, iota, N_EXP), axis=0, keepdims=True
            )
            vals_rows.append(mx)
            inds_rows.append(idx)
            work = jnp.where(iota == idx, -1.0, work)
        vals_ref[...] = jnp.concatenate(vals_rows, axis=0).T
        inds_ref[...] = jnp.concatenate(inds_rows, axis=0).T


@jax.jit
def kernel(hidden_states, W_gate, W_noise):
    del W_noise
    grid = (N_TOK // TB, ND)
    vals, inds, gates = pl.pallas_call(
        _router_block,
        grid=grid,
        in_specs=[
            pl.BlockSpec((TB, DC), lambda i, j: (i, j)),
            pl.BlockSpec((N_EXP, DC), lambda i, j: (0, j)),
        ],
        out_specs=[
            pl.BlockSpec((TB, TOP_K), lambda i, j: (i, 0)),
            pl.BlockSpec((TB, TOP_K), lambda i, j: (i, 0)),
            pl.BlockSpec((TB, N_EXP), lambda i, j: (i, 0)),
        ],
        out_shape=[
            jax.ShapeDtypeStruct((N_TOK, TOP_K), jnp.float32),
            jax.ShapeDtypeStruct((N_TOK, TOP_K), jnp.int32),
            jax.ShapeDtypeStruct((N_TOK, N_EXP), jnp.float32),
        ],
        scratch_shapes=[pltpu.VMEM((TB, N_EXP), jnp.float32)],
        compiler_params=pltpu.CompilerParams(
            dimension_semantics=("parallel", "arbitrary"),
        ),
    )(hidden_states, W_gate)
    return vals, inds, gates


# speedup vs baseline: 1.1847x; 1.1847x over previous
import functools

import jax
import jax.numpy as jnp
from jax.experimental import pallas as pl
from jax.experimental.pallas import tpu as pltpu

D = 2048
N_EXP = 64
TOP_K = 8
N_TOK = 16384

TB = 2048


def _router_block(xa_ref, xb_ref, w_ref, vals_ref, inds_ref, gates_ref):
    w = w_ref[...]
    dn = (((1,), (1,)), ((), ()))
    logits = jax.lax.dot_general(
        xa_ref[...], w[:, : D // 2], dn, preferred_element_type=jnp.float32
    ) + jax.lax.dot_general(
        xb_ref[...], w[:, D // 2 :], dn, preferred_element_type=jnp.float32
    )
    lt = logits.T
    m = jnp.max(lt, axis=0, keepdims=True)
    e = jnp.exp(lt - m)
    s = jnp.sum(e, axis=0, keepdims=True)
    gt = e / s
    gates_ref[...] = gt.T

    iota = jax.lax.broadcasted_iota(jnp.int32, (N_EXP, TB), 0)
    work = gt
    vals_rows = []
    inds_rows = []
    for _ in range(TOP_K):
        mx = jnp.max(work, axis=0, keepdims=True)
        idx = jnp.min(jnp.where(work == mx, iota, N_EXP), axis=0, keepdims=True)
        vals_rows.append(mx)
        inds_rows.append(idx)
        work = jnp.where(iota == idx, -1.0, work)
    vals_ref[...] = jnp.concatenate(vals_rows, axis=0).T
    inds_ref[...] = jnp.concatenate(inds_rows, axis=0).T


@jax.jit
def kernel(hidden_states, W_gate, W_noise):
    del W_noise
    grid = (N_TOK // TB,)
    vals, inds, gates = pl.pallas_call(
        _router_block,
        grid=grid,
        in_specs=[
            pl.BlockSpec((TB, D // 2), lambda i: (i, 0)),
            pl.BlockSpec((TB, D // 2), lambda i: (i, 1)),
            pl.BlockSpec((N_EXP, D), lambda i: (0, 0)),
        ],
        out_specs=[
            pl.BlockSpec((TB, TOP_K), lambda i: (i, 0)),
            pl.BlockSpec((TB, TOP_K), lambda i: (i, 0)),
            pl.BlockSpec((TB, N_EXP), lambda i: (i, 0)),
        ],
        out_shape=[
            jax.ShapeDtypeStruct((N_TOK, TOP_K), jnp.float32),
            jax.ShapeDtypeStruct((N_TOK, TOP_K), jnp.int32),
            jax.ShapeDtypeStruct((N_TOK, N_EXP), jnp.float32),
        ],
        compiler_params=pltpu.CompilerParams(
            dimension_semantics=("parallel",),
        ),
    )(hidden_states, hidden_states, W_gate)
    return vals, inds, gates
